# baseline (speedup 1.0000x reference)
import jax
import jax.numpy as jnp
from jax import lax
from jax.experimental import pallas as pl
from jax.experimental.pallas import tpu as pltpu
from jax.experimental.pallas import tpu_sc as plsc

ALPHA = 0.2
DAMP = 0.85
N = 10000
E = 320000
F = 128
NC = 2
NS = 16
NW = NC * NS
L = 16
EPW = E // NW
B = 80
NB = EPW // B
GPB = B // L
EPC = E // NS
CB = 2000
NCH = EPC // CB
RPT = N // NS
NP = 10240
BK = NP // NS
EPS = 1e-12

_mesh = plsc.VectorSubcoreMesh(
    core_axis_name="c", subcore_axis_name="s", num_cores=NC, num_subcores=NS)


def _pq_body(x_ref, w1_ref, w2_ref, p_ref, q_ref):
    xb = x_ref[...]
    p_ref[...] = jnp.dot(xb, w1_ref[...], preferred_element_type=jnp.float32)
    q_ref[...] = jnp.dot(xb, w2_ref[...], preferred_element_type=jnp.float32)


def _compute_pq(x, w1, w2):
    blk = 1000
    return pl.pallas_call(
        _pq_body,
        grid=(N // blk,),
        in_specs=[pl.BlockSpec((blk, F), lambda i: (i, 0)),
                  pl.BlockSpec((F, F), lambda i: (0, 0)),
                  pl.BlockSpec((F, F), lambda i: (0, 0))],
        out_specs=[pl.BlockSpec((blk, F), lambda i: (i, 0)),
                   pl.BlockSpec((blk, F), lambda i: (i, 0))],
        out_shape=[jax.ShapeDtypeStruct((N, F), jnp.float32)] * 2,
    )(x, w1, w2)


def _r_body(e_ref, w_ref, o_ref):
    o_ref[...] = jnp.dot(e_ref[...], w_ref[...],
                         preferred_element_type=jnp.float32)


def _compute_r(edge_embed, w3):
    blk = 4000
    nr = edge_embed.shape[1]
    return pl.pallas_call(
        _r_body,
        grid=(E // blk,),
        in_specs=[pl.BlockSpec((blk, nr), lambda i: (i, 0)),
                  pl.BlockSpec((nr, F), lambda i: (0, 0))],
        out_specs=pl.BlockSpec((blk, F), lambda i: (i, 0)),
        out_shape=jax.ShapeDtypeStruct((E, F), jnp.float32),
    )(edge_embed, w3)


def _fin_body(h0_ref, h1_ref, ers_ref, o_ref):
    hp = (h0_ref[...] + h1_ref[...]) / ers_ref[...]
    o_ref[...] = jnp.where(hp > 0.0, hp, jnp.exp(jnp.minimum(hp, 0.0)) - 1.0)


def _finalize(h0, h1, ers2d):
    blk = 1000
    return pl.pallas_call(
        _fin_body,
        grid=(N // blk,),
        in_specs=[pl.BlockSpec((blk, F), lambda i: (i, 0)),
                  pl.BlockSpec((blk, F), lambda i: (i, 0)),
                  pl.BlockSpec((blk, 1), lambda i: (i, 0))],
        out_specs=pl.BlockSpec((blk, F), lambda i: (i, 0)),
        out_shape=jax.ShapeDtypeStruct((N, F), jnp.float32),
    )(h0, h1, ers2d)


def _lane_total(v):
    idx = lax.broadcasted_iota(jnp.int32, (L,), 0)
    dnums = lax.GatherDimensionNumbers(
        offset_dims=(), collapsed_slice_dims=(0,), start_index_map=(0,))
    for k in (8, 4, 2, 1):
        perm = jnp.bitwise_xor(idx, k)
        v = v + lax.gather(v, perm[:, None], dnums, (1,),
                           mode=lax.GatherScatterMode.PROMISE_IN_BOUNDS)
    return v


def _zero_ref(ref, n):
    def _z(i, carry):
        ref[pl.ds(i * L, L)] = jnp.zeros((L,), jnp.float32)
        return carry
    lax.fori_loop(0, n // L, _z, 0)


def _acc_from(ref_dst, ref_src, n):
    def _a(i, carry):
        ref_dst[pl.ds(i * L, L)] = (ref_dst[pl.ds(i * L, L)]
                                    + ref_src[pl.ds(i * L, L)])
        return carry
    lax.fori_loop(0, n // L, _a, 0)


def _fix_zeros(ref, n):
    def _f(i, carry):
        v = ref[pl.ds(i * L, L)]
        ref[pl.ds(i * L, L)] = jnp.where(v == 0.0, EPS, v)
        return carry
    lax.fori_loop(0, n // L, _f, 0)


def _k1_body(p_hbm, q_hbm, r_hbm, src_hbm, dst_hbm, a2_hbm,
             em_hbm, ee_hbm, ers_hbm,
             srcall, dstall, eeall, erslocal, a2buf,
             pbufs, qbufs, rbufs, embufs,
             semp, semq, semr, seme):
    wid = lax.axis_index("s") * NC + lax.axis_index("c")
    base = wid * EPW
    pltpu.sync_copy(src_hbm.at[pl.ds(base, EPW)], srcall)
    pltpu.sync_copy(dst_hbm.at[pl.ds(base, EPW)], dstall)
    pltpu.sync_copy(a2_hbm, a2buf)
    _zero_ref(erslocal, NP)
    _zero_ref(eeall, EPW)
    a2c = [a2buf[pl.ds(ch * L, L)] for ch in range(F // L)]
    pb = [pbufs.at[0], pbufs.at[1]]
    qb = [qbufs.at[0], qbufs.at[1]]
    rb = [rbufs.at[0], rbufs.at[1]]
    emb = [embufs.at[0], embufs.at[1]]
    sp = [semp.at[0], semp.at[1]]
    sq = [semq.at[0], semq.at[1]]
    sr = [semr.at[0], semr.at[1]]
    se = [seme.at[0], seme.at[1]]

    def _issue(b, k):
        lo = b * B
        pltpu.async_copy(p_hbm.at[srcall.at[pl.ds(lo, B)]], pb[k], sp[k])
        pltpu.async_copy(q_hbm.at[dstall.at[pl.ds(lo, B)]], qb[k], sq[k])
        pltpu.async_copy(r_hbm.at[pl.ds(base + lo, B)], rb[k], sr[k])

    def _wait_in(k):
        pltpu.make_async_copy(p_hbm.at[pl.ds(0, B)], pb[k], sp[k]).wait()
        pltpu.make_async_copy(q_hbm.at[pl.ds(0, B)], qb[k], sq[k]).wait()
        pltpu.make_async_copy(r_hbm.at[pl.ds(0, B)], rb[k], sr[k]).wait()

    def _compute(b, k):
        lo = b * B

        def _edge(e, cc):
            acc = jnp.zeros((L,), jnp.float32)
            for ch in range(F // L):
                pv = pb[k][e, pl.ds(ch * L, L)]
                qv = qb[k][e, pl.ds(ch * L, L)]
                rv = rb[k][e, pl.ds(ch * L, L)]
                m = pv + qv + rv
                emv = jnp.maximum(m, ALPHA * m)
                emb[k][e, pl.ds(ch * L, L)] = emv
                acc = acc + a2c[ch] * emv
            plsc.addupdate_scatter(eeall, [jnp.full((L,), lo + e,
                                                     jnp.int32)], acc)
            return cc

        lax.fori_loop(0, B, _edge, 0)
        for g in range(GPB):
            t = eeall[pl.ds(lo + g * L, L)]
            powers = -jnp.maximum(t, ALPHA * t)
            ev = jnp.exp(powers)
            eeall[pl.ds(lo + g * L, L)] = ev
            dl = dstall[pl.ds(lo + g * L, L)]
            plsc.addupdate_scatter(erslocal, [dl], ev)
        pltpu.async_copy(emb[k], em_hbm.at[pl.ds(base + lo, B)], se[k])

    _issue(0, 0)
    _wait_in(0)

    def _pair(j, carry):
        for k in range(2):
            b = 2 * j + k

            @pl.when(b + 1 < NB)
            def _():
                _issue(b + 1, 1 - k)

            @pl.when(b >= 2)
            def _():
                pltpu.make_async_copy(p_hbm.at[pl.ds(0, B)], emb[k],
                                      se[k]).wait()
            _compute(b, k)

            @pl.when(b + 1 < NB)
            def _():
                _wait_in(1 - k)
        return carry

    lax.fori_loop(0, NB // 2, _pair, 0)
    pltpu.make_async_copy(p_hbm.at[pl.ds(0, B)], emb[0], se[0]).wait()
    _compute(NB - 1, 0)
    pltpu.make_async_copy(p_hbm.at[pl.ds(0, B)], emb[0], se[0]).wait()
    pltpu.make_async_copy(p_hbm.at[pl.ds(0, B)], emb[1], se[1]).wait()
    pltpu.sync_copy(eeall, ee_hbm.at[pl.ds(base, EPW)])
    pltpu.sync_copy(erslocal, ers_hbm.at[pl.ds(wid * NP, NP)])


def _sc_pass1(p, q, r, src, dst, a2v):
    kern = pl.kernel(
        _k1_body,
        out_type=[jax.ShapeDtypeStruct((E, F), jnp.float32),
                  jax.ShapeDtypeStruct((E,), jnp.float32),
                  jax.ShapeDtypeStruct((NW * NP,), jnp.float32)],
        mesh=_mesh,
        compiler_params=pltpu.CompilerParams(needs_layout_passes=False),
        scratch_types=[pltpu.VMEM((EPW,), jnp.int32),
                       pltpu.VMEM((EPW,), jnp.int32),
                       pltpu.VMEM((EPW,), jnp.float32),
                       pltpu.VMEM((NP,), jnp.float32),
                       pltpu.VMEM((F,), jnp.float32),
                       pltpu.VMEM((2, B, F), jnp.float32),
                       pltpu.VMEM((2, B, F), jnp.float32),
                       pltpu.VMEM((2, B, F), jnp.float32),
                       pltpu.VMEM((2, B, F), jnp.float32),
                       pltpu.SemaphoreType.DMA((2,)),
                       pltpu.SemaphoreType.DMA((2,)),
                       pltpu.SemaphoreType.DMA((2,)),
                       pltpu.SemaphoreType.DMA((2,))],
    )
    return kern(p, q, r, src, dst, a2v)


def _acc_rows(dst, ref2d, nrows, n):
    def _a(i, carry):
        v = ref2d[0, pl.ds(i * L, L)]
        for j in range(1, nrows):
            v = v + ref2d[j, pl.ds(i * L, L)]
        dst[pl.ds(i * L, L)] = v
        return carry
    lax.fori_loop(0, n // L, _a, 0)


def _k2_body(ersp_hbm, ee_hbm, src_hbm, dst_hbm, er_hbm,
             ern_hbm, sm_hbm, ers_hbm,
             ers, erb, den, num, smb, blk, srcb, dstb, eb, cb640,
             sharedA, sharedB, semb):
    c = lax.axis_index("c")
    s = lax.axis_index("s")
    pltpu.sync_copy(er_hbm, erb)

    for j in range(NW):
        pltpu.async_copy(ersp_hbm.at[pl.ds(j * NP + s * BK, BK)], blk.at[j],
                         semb)
    for j in range(NW):
        pltpu.make_async_copy(ersp_hbm.at[pl.ds(0, BK)], blk.at[j],
                              semb).wait()
    _acc_rows(cb640, blk, NW, BK)
    _fix_zeros(cb640, BK)
    pltpu.sync_copy(cb640, sharedB.at[pl.ds(s * BK, BK)])
    plsc.subcore_barrier()
    pltpu.sync_copy(sharedB, ers)
    plsc.subcore_barrier()

    ebase = s * EPC
    _zero_ref(den, NP)

    def _chunk_a(k, carry):
        off = ebase + k * CB
        pltpu.sync_copy(src_hbm.at[pl.ds(off, CB)], srcb)
        pltpu.sync_copy(ee_hbm.at[pl.ds(off, CB)], eb)

        def _grp(g, cc):
            sl = srcb[pl.ds(g * L, L)]
            ev = eb[pl.ds(g * L, L)]
            esrc = plsc.load_gather(ers, [sl])
            rav = ev / esrc
            plsc.addupdate_scatter(den, [sl], rav)
            return cc

        lax.fori_loop(0, CB // L, _grp, 0)
        return carry

    lax.fori_loop(0, NCH, _chunk_a, 0)

    pltpu.sync_copy(den, sharedA.at[pl.ds(s * NP, NP)])
    plsc.subcore_barrier()
    for j in range(NS):
        pltpu.async_copy(sharedA.at[pl.ds(j * NP + s * BK, BK)], blk.at[j],
                         semb)
    for j in range(NS):
        pltpu.make_async_copy(ersp_hbm.at[pl.ds(0, BK)], blk.at[j],
                              semb).wait()
    _acc_rows(cb640, blk, NS, BK)
    _fix_zeros(cb640, BK)
    pltpu.sync_copy(cb640, sharedB.at[pl.ds(s * BK, BK)])
    plsc.subcore_barrier()
    pltpu.sync_copy(sharedB, den)
    plsc.subcore_barrier()

    _zero_ref(num, NP)

    def _chunk_b(k, carry):
        off = ebase + k * CB
        pltpu.sync_copy(src_hbm.at[pl.ds(off, CB)], srcb)
        pltpu.sync_copy(dst_hbm.at[pl.ds(off, CB)], dstb)
        pltpu.sync_copy(ee_hbm.at[pl.ds(off, CB)], eb)

        def _grp(g, cc):
            sl = srcb[pl.ds(g * L, L)]
            dl = dstb[pl.ds(g * L, L)]
            ev = eb[pl.ds(g * L, L)]
            esrc = plsc.load_gather(ers, [sl])
            rav = ev / esrc
            densrc = plsc.load_gather(den, [sl])
            ersrc = plsc.load_gather(erb, [sl])
            nv = rav * ersrc / densrc
            plsc.addupdate_scatter(num, [dl], nv)
            return cc

        lax.fori_loop(0, CB // L, _grp, 0)
        return carry

    lax.fori_loop(0, NCH, _chunk_b, 0)

    pltpu.sync_copy(num, sharedA.at[pl.ds(s * NP, NP)])
    plsc.subcore_barrier()
    for j in range(NS):
        pltpu.async_copy(sharedA.at[pl.ds(j * NP + s * BK, BK)], blk.at[j],
                         semb)
    for j in range(NS):
        pltpu.make_async_copy(ersp_hbm.at[pl.ds(0, BK)], blk.at[j],
                              semb).wait()
    _acc_rows(cb640, blk, NS, BK)
    pltpu.sync_copy(cb640, sharedB.at[pl.ds(s * BK, BK)])
    plsc.subcore_barrier()
    pltpu.sync_copy(sharedB, num)

    def _ern(i, carry):
        v = num[pl.ds(i * L, L)]
        num[pl.ds(i * L, L)] = (1.0 - DAMP) + DAMP * v
        return carry
    lax.fori_loop(0, N // L, _ern, 0)

    def _mx(i, acc):
        return jnp.maximum(acc, num[pl.ds(i * L, L)])
    mx = jnp.max(lax.fori_loop(0, N // L, _mx,
                               jnp.full((L,), -jnp.inf, jnp.float32)))

    def _se(i, acc):
        return acc + jnp.exp(num[pl.ds(i * L, L)] - mx)
    ssum = jnp.sum(lax.fori_loop(0, N // L, _se,
                                 jnp.zeros((L,), jnp.float32)))

    def _sm(i, carry):
        smb[pl.ds(i * L, L)] = jnp.exp(num[pl.ds(i * L, L)] - mx) / ssum
        return carry
    lax.fori_loop(0, N // L, _sm, 0)

    @pl.when(jnp.logical_and(c == 0, s == 0))
    def _write():
        pltpu.sync_copy(num.at[pl.ds(0, N)], ern_hbm)
        pltpu.sync_copy(smb, sm_hbm)
        pltpu.sync_copy(ers.at[pl.ds(0, N)], ers_hbm)


def _sc_pass2(ersp, ee, src, dst, entity_rank):
    kern = pl.kernel(
        _k2_body,
        out_type=[jax.ShapeDtypeStruct((N,), jnp.float32),
                  jax.ShapeDtypeStruct((N,), jnp.float32),
                  jax.ShapeDtypeStruct((N,), jnp.float32)],
        mesh=_mesh,
        compiler_params=pltpu.CompilerParams(needs_layout_passes=False),
        scratch_types=[pltpu.VMEM((NP,), jnp.float32),
                       pltpu.VMEM((N,), jnp.float32),
                       pltpu.VMEM((NP,), jnp.float32),
                       pltpu.VMEM((NP,), jnp.float32),
                       pltpu.VMEM((N,), jnp.float32),
                       pltpu.VMEM((NW, BK), jnp.float32),
                       pltpu.VMEM((CB,), jnp.int32),
                       pltpu.VMEM((CB,), jnp.int32),
                       pltpu.VMEM((CB,), jnp.float32),
                       pltpu.VMEM((BK,), jnp.float32),
                       pltpu.VMEM_SHARED((NS * NP,), jnp.float32),
                       pltpu.VMEM_SHARED((NP,), jnp.float32),
                       pltpu.SemaphoreType.DMA],
    )
    return kern(ersp, ee, src, dst, entity_rank)


def _k3_body(em_hbm, ee_hbm, src_hbm, dst_hbm, sm_hbm,
             hp_hbm,
             srcb, eb, coeffall, smb, didxs, embufs, h_sh,
             seme, semi, sems):
    c = lax.axis_index("c")
    s = lax.axis_index("s")
    wid = s * NC + c
    base = wid * EPW
    pltpu.sync_copy(sm_hbm, smb)

    def _cchunk(k2, carry):
        off = k2 * CB
        pltpu.sync_copy(src_hbm.at[pl.ds(base + off, CB)], srcb)
        pltpu.sync_copy(ee_hbm.at[pl.ds(base + off, CB)], eb)

        def _grp(g, cc):
            sl = srcb[pl.ds(g * L, L)]
            smv = plsc.load_gather(smb, [sl])
            ev = eb[pl.ds(g * L, L)]
            coeffall[pl.ds(off + g * L, L)] = smv * ev
            return cc

        lax.fori_loop(0, CB // L, _grp, 0)
        return carry

    lax.fori_loop(0, EPW // CB, _cchunk, 0)

    def _zz(e, cc):
        for ch in range(F // L):
            embufs[0, e, pl.ds(ch * L, L)] = jnp.zeros((L,), jnp.float32)
        return cc
    lax.fori_loop(0, B, _zz, 0)
    rows0 = s * 640
    nch = jnp.where(s == NS - 1, 5, 8)

    def _zc(i, cc):
        pltpu.sync_copy(embufs.at[0], h_sh.at[pl.ds(rows0 + i * B, B)])
        return cc
    lax.fori_loop(0, nch, _zc, 0)
    plsc.subcore_barrier()

    emb = [embufs.at[0], embufs.at[1]]
    se = [seme.at[0], seme.at[1]]
    si = [semi.at[0], semi.at[1]]
    ss = [sems.at[0], sems.at[1]]

    def _issue(b, k):
        pltpu.async_copy(em_hbm.at[pl.ds(base + b * B, B)], emb[k], se[k])
        pltpu.async_copy(dst_hbm.at[pl.ds(base + b * B, B)], didxs.at[k],
                         si[k])

    def _compute(b, k):
        lo = b * B
        pltpu.make_async_copy(em_hbm.at[pl.ds(0, B)], emb[k], se[k]).wait()
        pltpu.make_async_copy(dst_hbm.at[pl.ds(0, B)], didxs.at[k],
                              si[k]).wait()

        def _edge(e, cc):
            cs = plsc.load_gather(coeffall, [jnp.full((L,), lo + e,
                                                      jnp.int32)])
            for ch in range(F // L):
                embufs[k, e, pl.ds(ch * L, L)] = (
                    embufs[k, e, pl.ds(ch * L, L)] * cs)
            return cc

        lax.fori_loop(0, B, _edge, 0)
        pltpu.async_copy(emb[k], h_sh.at[didxs.at[k]], ss[k], add=True)

    def _wait_scat(k):
        pltpu.make_async_copy(em_hbm.at[pl.ds(0, B)], emb[k], ss[k]).wait()

    _issue(0, 0)

    def _pair(j, carry):
        for k in range(2):
            b = 2 * j + k

            @pl.when(b >= 1)
            def _():
                _wait_scat(1 - k)

            @pl.when(b + 1 < NB)
            def _():
                _issue(b + 1, 1 - k)
            _compute(b, k)
        return carry

    lax.fori_loop(0, NB // 2, _pair, 0)
    _wait_scat(1)
    _compute(NB - 1, 0)
    _wait_scat(0)
    plsc.subcore_barrier()

    def _wc(i, cc):
        pltpu.sync_copy(h_sh.at[pl.ds(rows0 + i * B, B)],
                        hp_hbm.at[pl.ds(c * N + rows0 + i * B, B)])
        return cc
    lax.fori_loop(0, nch, _wc, 0)


def _sc_pass3(em, ee, src, dst, sm):
    kern = pl.kernel(
        _k3_body,
        out_type=jax.ShapeDtypeStruct((NC * N, F), jnp.float32),
        mesh=_mesh,
        compiler_params=pltpu.CompilerParams(needs_layout_passes=False),
        scratch_types=[pltpu.VMEM((CB,), jnp.int32),
                       pltpu.VMEM((CB,), jnp.float32),
                       pltpu.VMEM((EPW,), jnp.float32),
                       pltpu.VMEM((N,), jnp.float32),
                       pltpu.VMEM((2, B), jnp.int32),
                       pltpu.VMEM((2, B, F), jnp.float32),
                       pltpu.VMEM_SHARED((N, F), jnp.float32),
                       pltpu.SemaphoreType.DMA((2,)),
                       pltpu.SemaphoreType.DMA((2,)),
                       pltpu.SemaphoreType.DMA((2,))],
    )
    return kern(em, ee, src, dst, sm)


def kernel(input, edge, edge_embed, edge_list_nhop, edge_embed_nhop,
           confidence, entity_rank, Corpus_, a, a_2):
    x = input
    src = edge[0]
    dst = edge[1]
    w1 = a[:, :F].T
    w2 = a[:, F:2 * F].T
    w3 = a[:, 2 * F:].T
    a2v = a_2[0]
    p, q = _compute_pq(x, w1, w2)
    r = _compute_r(edge_embed, w3)
    em, ee, ersp = _sc_pass1(p, q, r, src, dst, a2v)
    ern, sm, ers = _sc_pass2(ersp, ee, src, dst, entity_rank)
    hp = _sc_pass3(em, ee, src, dst, sm)
    h = _finalize(hp[:N], hp[N:], ers[:, None])
    return (h, ern)


# speedup vs baseline: 1.0866x; 1.0866x over previous
import jax
import jax.numpy as jnp
from jax import lax
from jax.experimental import pallas as pl
from jax.experimental.pallas import tpu as pltpu
from jax.experimental.pallas import tpu_sc as plsc

ALPHA = 0.2
DAMP = 0.85
N = 10000
E = 320000
F = 128
NC = 2
NS = 16
NW = NC * NS
L = 16
EPW = E // NW
B = 80
NB = EPW // B
GPB = B // L
EPC = E // NS
CB = 2000
NCH = EPC // CB
RPT = N // NS
NP = 10240
BK = NP // NS
EPS = 1e-12

_mesh = plsc.VectorSubcoreMesh(
    core_axis_name="c", subcore_axis_name="s", num_cores=NC, num_subcores=NS)


def _pq_body(x_ref, w1_ref, w2_ref, p_ref, q_ref):
    xb = x_ref[...]
    p_ref[...] = jnp.dot(xb, w1_ref[...], preferred_element_type=jnp.float32)
    q_ref[...] = jnp.dot(xb, w2_ref[...], preferred_element_type=jnp.float32)


def _compute_pq(x, w1, w2):
    blk = 1000
    return pl.pallas_call(
        _pq_body,
        grid=(N // blk,),
        in_specs=[pl.BlockSpec((blk, F), lambda i: (i, 0)),
                  pl.BlockSpec((F, F), lambda i: (0, 0)),
                  pl.BlockSpec((F, F), lambda i: (0, 0))],
        out_specs=[pl.BlockSpec((blk, F), lambda i: (i, 0)),
                   pl.BlockSpec((blk, F), lambda i: (i, 0))],
        out_shape=[jax.ShapeDtypeStruct((N, F), jnp.float32)] * 2,
    )(x, w1, w2)


def _r_body(e_ref, w_ref, o_ref):
    o_ref[...] = jnp.dot(e_ref[...], w_ref[...],
                         preferred_element_type=jnp.float32)


def _compute_r(edge_embed, w3):
    blk = 4000
    nr = edge_embed.shape[1]
    return pl.pallas_call(
        _r_body,
        grid=(E // blk,),
        in_specs=[pl.BlockSpec((blk, nr), lambda i: (i, 0)),
                  pl.BlockSpec((nr, F), lambda i: (0, 0))],
        out_specs=pl.BlockSpec((blk, F), lambda i: (i, 0)),
        out_shape=jax.ShapeDtypeStruct((E, F), jnp.float32),
    )(edge_embed, w3)


def _fin_body(h0_ref, h1_ref, ers_ref, o_ref):
    hp = (h0_ref[...] + h1_ref[...]) / ers_ref[...]
    o_ref[...] = jnp.where(hp > 0.0, hp, jnp.exp(jnp.minimum(hp, 0.0)) - 1.0)


def _finalize(h0, h1, ers2d):
    blk = 1000
    return pl.pallas_call(
        _fin_body,
        grid=(N // blk,),
        in_specs=[pl.BlockSpec((blk, F), lambda i: (i, 0)),
                  pl.BlockSpec((blk, F), lambda i: (i, 0)),
                  pl.BlockSpec((blk, 1), lambda i: (i, 0))],
        out_specs=pl.BlockSpec((blk, F), lambda i: (i, 0)),
        out_shape=jax.ShapeDtypeStruct((N, F), jnp.float32),
    )(h0, h1, ers2d)


def _lane_total(v):
    idx = lax.broadcasted_iota(jnp.int32, (L,), 0)
    dnums = lax.GatherDimensionNumbers(
        offset_dims=(), collapsed_slice_dims=(0,), start_index_map=(0,))
    for k in (8, 4, 2, 1):
        perm = jnp.bitwise_xor(idx, k)
        v = v + lax.gather(v, perm[:, None], dnums, (1,),
                           mode=lax.GatherScatterMode.PROMISE_IN_BOUNDS)
    return v


def _zero_ref(ref, n):
    def _z(i, carry):
        ref[pl.ds(i * L, L)] = jnp.zeros((L,), jnp.float32)
        return carry
    lax.fori_loop(0, n // L, _z, 0)


def _acc_from(ref_dst, ref_src, n):
    def _a(i, carry):
        ref_dst[pl.ds(i * L, L)] = (ref_dst[pl.ds(i * L, L)]
                                    + ref_src[pl.ds(i * L, L)])
        return carry
    lax.fori_loop(0, n // L, _a, 0)


def _fix_zeros(ref, n):
    def _f(i, carry):
        v = ref[pl.ds(i * L, L)]
        ref[pl.ds(i * L, L)] = jnp.where(v == 0.0, EPS, v)
        return carry
    lax.fori_loop(0, n // L, _f, 0)


def _k1_body(p_hbm, q_hbm, r_hbm, src_hbm, dst_hbm, a2_hbm,
             em_hbm, ee_hbm, ers_hbm,
             srcall, dstall, eeall, erslocal, a2buf,
             pbufs, qbufs, rbufs, embufs,
             semp, semq, semr, seme):
    wid = lax.axis_index("s") * NC + lax.axis_index("c")
    base = wid * EPW
    pltpu.sync_copy(src_hbm.at[pl.ds(base, EPW)], srcall)
    pltpu.sync_copy(dst_hbm.at[pl.ds(base, EPW)], dstall)
    pltpu.sync_copy(a2_hbm, a2buf)
    _zero_ref(erslocal, NP)
    a2c = [a2buf[pl.ds(ch * L, L)] for ch in range(F // L)]
    pb = [pbufs.at[0], pbufs.at[1]]
    qb = [qbufs.at[0], qbufs.at[1]]
    rb = [rbufs.at[0], rbufs.at[1]]
    emb = [embufs.at[0], embufs.at[1]]
    sp = [semp.at[0], semp.at[1]]
    sq = [semq.at[0], semq.at[1]]
    sr = [semr.at[0], semr.at[1]]
    se = [seme.at[0], seme.at[1]]

    def _issue(b, k):
        lo = b * B
        pltpu.async_copy(p_hbm.at[srcall.at[pl.ds(lo, B)]], pb[k], sp[k])
        pltpu.async_copy(q_hbm.at[dstall.at[pl.ds(lo, B)]], qb[k], sq[k])
        pltpu.async_copy(r_hbm.at[pl.ds(base + lo, B)], rb[k], sr[k])

    def _wait_in(k):
        pltpu.make_async_copy(p_hbm.at[pl.ds(0, B)], pb[k], sp[k]).wait()
        pltpu.make_async_copy(q_hbm.at[pl.ds(0, B)], qb[k], sq[k]).wait()
        pltpu.make_async_copy(r_hbm.at[pl.ds(0, B)], rb[k], sr[k]).wait()

    def _compute(b, k):
        lo = b * B

        def _edge(e, cc):
            acc = jnp.zeros((L,), jnp.float32)
            for ch in range(F // L):
                pv = pb[k][e, pl.ds(ch * L, L)]
                qv = qb[k][e, pl.ds(ch * L, L)]
                rv = rb[k][e, pl.ds(ch * L, L)]
                m = pv + qv + rv
                emv = jnp.maximum(m, ALPHA * m)
                emb[k][e, pl.ds(ch * L, L)] = emv
                acc = acc + a2c[ch] * emv
            t = _lane_total(acc)
            plsc.store_scatter(eeall, [jnp.full((L,), lo + e, jnp.int32)], t)
            return cc

        lax.fori_loop(0, B, _edge, 0)
        for g in range(GPB):
            t = eeall[pl.ds(lo + g * L, L)]
            powers = -jnp.maximum(t, ALPHA * t)
            ev = jnp.exp(powers)
            eeall[pl.ds(lo + g * L, L)] = ev
            dl = dstall[pl.ds(lo + g * L, L)]
            plsc.addupdate_scatter(erslocal, [dl], ev)
        pltpu.async_copy(emb[k], em_hbm.at[pl.ds(base + lo, B)], se[k])

    _issue(0, 0)
    _wait_in(0)

    def _pair(j, carry):
        for k in range(2):
            b = 2 * j + k

            @pl.when(b + 1 < NB)
            def _():
                _issue(b + 1, 1 - k)

            @pl.when(b >= 2)
            def _():
                pltpu.make_async_copy(p_hbm.at[pl.ds(0, B)], emb[k],
                                      se[k]).wait()
            _compute(b, k)

            @pl.when(b + 1 < NB)
            def _():
                _wait_in(1 - k)
        return carry

    lax.fori_loop(0, NB // 2, _pair, 0)
    pltpu.make_async_copy(p_hbm.at[pl.ds(0, B)], emb[0], se[0]).wait()
    _compute(NB - 1, 0)
    pltpu.make_async_copy(p_hbm.at[pl.ds(0, B)], emb[0], se[0]).wait()
    pltpu.make_async_copy(p_hbm.at[pl.ds(0, B)], emb[1], se[1]).wait()
    pltpu.sync_copy(eeall, ee_hbm.at[pl.ds(base, EPW)])
    pltpu.sync_copy(erslocal, ers_hbm.at[pl.ds(wid * NP, NP)])


def _sc_pass1(p, q, r, src, dst, a2v):
    kern = pl.kernel(
        _k1_body,
        out_type=[jax.ShapeDtypeStruct((E, F), jnp.float32),
                  jax.ShapeDtypeStruct((E,), jnp.float32),
                  jax.ShapeDtypeStruct((NW * NP,), jnp.float32)],
        mesh=_mesh,
        compiler_params=pltpu.CompilerParams(needs_layout_passes=False),
        scratch_types=[pltpu.VMEM((EPW,), jnp.int32),
                       pltpu.VMEM((EPW,), jnp.int32),
                       pltpu.VMEM((EPW,), jnp.float32),
                       pltpu.VMEM((NP,), jnp.float32),
                       pltpu.VMEM((F,), jnp.float32),
                       pltpu.VMEM((2, B, F), jnp.float32),
                       pltpu.VMEM((2, B, F), jnp.float32),
                       pltpu.VMEM((2, B, F), jnp.float32),
                       pltpu.VMEM((2, B, F), jnp.float32),
                       pltpu.SemaphoreType.DMA((2,)),
                       pltpu.SemaphoreType.DMA((2,)),
                       pltpu.SemaphoreType.DMA((2,)),
                       pltpu.SemaphoreType.DMA((2,))],
    )
    return kern(p, q, r, src, dst, a2v)


def _acc_rows(dst, ref2d, nrows, n):
    def _a(i, carry):
        v = ref2d[0, pl.ds(i * L, L)]
        for j in range(1, nrows):
            v = v + ref2d[j, pl.ds(i * L, L)]
        dst[pl.ds(i * L, L)] = v
        return carry
    lax.fori_loop(0, n // L, _a, 0)


def _k2_body(ersp_hbm, ee_hbm, src_hbm, dst_hbm, er_hbm,
             ern_hbm, sm_hbm, ers_hbm,
             ers, erb, den, num, smb, blk, srcb, dstb, eb, cb640,
             sharedA, sharedB, semb):
    c = lax.axis_index("c")
    s = lax.axis_index("s")
    pltpu.sync_copy(er_hbm, erb)

    for j in range(NW):
        pltpu.async_copy(ersp_hbm.at[pl.ds(j * NP + s * BK, BK)], blk.at[j],
                         semb)
    for j in range(NW):
        pltpu.make_async_copy(ersp_hbm.at[pl.ds(0, BK)], blk.at[j],
                              semb).wait()
    _acc_rows(cb640, blk, NW, BK)
    _fix_zeros(cb640, BK)
    pltpu.sync_copy(cb640, sharedB.at[pl.ds(s * BK, BK)])
    plsc.subcore_barrier()
    pltpu.sync_copy(sharedB, ers)
    plsc.subcore_barrier()

    ebase = s * EPC
    _zero_ref(den, NP)

    def _chunk_a(k, carry):
        off = ebase + k * CB
        pltpu.sync_copy(src_hbm.at[pl.ds(off, CB)], srcb)
        pltpu.sync_copy(ee_hbm.at[pl.ds(off, CB)], eb)

        def _grp(g, cc):
            sl = srcb[pl.ds(g * L, L)]
            ev = eb[pl.ds(g * L, L)]
            esrc = plsc.load_gather(ers, [sl])
            rav = ev / esrc
            plsc.addupdate_scatter(den, [sl], rav)
            return cc

        lax.fori_loop(0, CB // L, _grp, 0)
        return carry

    lax.fori_loop(0, NCH, _chunk_a, 0)

    pltpu.sync_copy(den, sharedA.at[pl.ds(s * NP, NP)])
    plsc.subcore_barrier()
    for j in range(NS):
        pltpu.async_copy(sharedA.at[pl.ds(j * NP + s * BK, BK)], blk.at[j],
                         semb)
    for j in range(NS):
        pltpu.make_async_copy(ersp_hbm.at[pl.ds(0, BK)], blk.at[j],
                              semb).wait()
    _acc_rows(cb640, blk, NS, BK)
    _fix_zeros(cb640, BK)
    pltpu.sync_copy(cb640, sharedB.at[pl.ds(s * BK, BK)])
    plsc.subcore_barrier()
    pltpu.sync_copy(sharedB, den)
    plsc.subcore_barrier()

    _zero_ref(num, NP)

    def _chunk_b(k, carry):
        off = ebase + k * CB
        pltpu.sync_copy(src_hbm.at[pl.ds(off, CB)], srcb)
        pltpu.sync_copy(dst_hbm.at[pl.ds(off, CB)], dstb)
        pltpu.sync_copy(ee_hbm.at[pl.ds(off, CB)], eb)

        def _grp(g, cc):
            sl = srcb[pl.ds(g * L, L)]
            dl = dstb[pl.ds(g * L, L)]
            ev = eb[pl.ds(g * L, L)]
            esrc = plsc.load_gather(ers, [sl])
            rav = ev / esrc
            densrc = plsc.load_gather(den, [sl])
            ersrc = plsc.load_gather(erb, [sl])
            nv = rav * ersrc / densrc
            plsc.addupdate_scatter(num, [dl], nv)
            return cc

        lax.fori_loop(0, CB // L, _grp, 0)
        return carry

    lax.fori_loop(0, NCH, _chunk_b, 0)

    pltpu.sync_copy(num, sharedA.at[pl.ds(s * NP, NP)])
    plsc.subcore_barrier()
    for j in range(NS):
        pltpu.async_copy(sharedA.at[pl.ds(j * NP + s * BK, BK)], blk.at[j],
                         semb)
    for j in range(NS):
        pltpu.make_async_copy(ersp_hbm.at[pl.ds(0, BK)], blk.at[j],
                              semb).wait()
    _acc_rows(cb640, blk, NS, BK)
    pltpu.sync_copy(cb640, sharedB.at[pl.ds(s * BK, BK)])
    plsc.subcore_barrier()
    pltpu.sync_copy(sharedB, num)

    def _ern(i, carry):
        v = num[pl.ds(i * L, L)]
        num[pl.ds(i * L, L)] = (1.0 - DAMP) + DAMP * v
        return carry
    lax.fori_loop(0, N // L, _ern, 0)

    def _mx(i, acc):
        return jnp.maximum(acc, num[pl.ds(i * L, L)])
    mx = jnp.max(lax.fori_loop(0, N // L, _mx,
                               jnp.full((L,), -jnp.inf, jnp.float32)))

    def _se(i, acc):
        return acc + jnp.exp(num[pl.ds(i * L, L)] - mx)
    ssum = jnp.sum(lax.fori_loop(0, N // L, _se,
                                 jnp.zeros((L,), jnp.float32)))

    def _sm(i, carry):
        smb[pl.ds(i * L, L)] = jnp.exp(num[pl.ds(i * L, L)] - mx) / ssum
        return carry
    lax.fori_loop(0, N // L, _sm, 0)

    @pl.when(jnp.logical_and(c == 0, s == 0))
    def _write():
        pltpu.sync_copy(num.at[pl.ds(0, N)], ern_hbm)
        pltpu.sync_copy(smb, sm_hbm)
        pltpu.sync_copy(ers.at[pl.ds(0, N)], ers_hbm)


def _sc_pass2(ersp, ee, src, dst, entity_rank):
    kern = pl.kernel(
        _k2_body,
        out_type=[jax.ShapeDtypeStruct((N,), jnp.float32),
                  jax.ShapeDtypeStruct((N,), jnp.float32),
                  jax.ShapeDtypeStruct((N,), jnp.float32)],
        mesh=_mesh,
        compiler_params=pltpu.CompilerParams(needs_layout_passes=False),
        scratch_types=[pltpu.VMEM((NP,), jnp.float32),
                       pltpu.VMEM((N,), jnp.float32),
                       pltpu.VMEM((NP,), jnp.float32),
                       pltpu.VMEM((NP,), jnp.float32),
                       pltpu.VMEM((N,), jnp.float32),
                       pltpu.VMEM((NW, BK), jnp.float32),
                       pltpu.VMEM((CB,), jnp.int32),
                       pltpu.VMEM((CB,), jnp.int32),
                       pltpu.VMEM((CB,), jnp.float32),
                       pltpu.VMEM((BK,), jnp.float32),
                       pltpu.VMEM_SHARED((NS * NP,), jnp.float32),
                       pltpu.VMEM_SHARED((NP,), jnp.float32),
                       pltpu.SemaphoreType.DMA],
    )
    return kern(ersp, ee, src, dst, entity_rank)


def _k3_body(em_hbm, ee_hbm, src_hbm, dst_hbm, sm_hbm,
             hp_hbm,
             srcb, eb, coeffall, smb, didxs, embufs, h_sh,
             seme, semi, sems):
    c = lax.axis_index("c")
    s = lax.axis_index("s")
    wid = s * NC + c
    base = wid * EPW
    pltpu.sync_copy(sm_hbm, smb)

    def _cchunk(k2, carry):
        off = k2 * CB
        pltpu.sync_copy(src_hbm.at[pl.ds(base + off, CB)], srcb)
        pltpu.sync_copy(ee_hbm.at[pl.ds(base + off, CB)], eb)

        def _grp(g, cc):
            sl = srcb[pl.ds(g * L, L)]
            smv = plsc.load_gather(smb, [sl])
            ev = eb[pl.ds(g * L, L)]
            coeffall[pl.ds(off + g * L, L)] = smv * ev
            return cc

        lax.fori_loop(0, CB // L, _grp, 0)
        return carry

    lax.fori_loop(0, EPW // CB, _cchunk, 0)

    def _zz(e, cc):
        for ch in range(F // L):
            embufs[0, e, pl.ds(ch * L, L)] = jnp.zeros((L,), jnp.float32)
        return cc
    lax.fori_loop(0, B, _zz, 0)
    rows0 = s * 640
    nch = jnp.where(s == NS - 1, 5, 8)

    def _zc(i, cc):
        pltpu.sync_copy(embufs.at[0], h_sh.at[pl.ds(rows0 + i * B, B)])
        return cc
    lax.fori_loop(0, nch, _zc, 0)
    plsc.subcore_barrier()

    emb = [embufs.at[0], embufs.at[1]]
    se = [seme.at[0], seme.at[1]]
    si = [semi.at[0], semi.at[1]]
    ss = [sems.at[0], sems.at[1]]

    def _issue(b, k):
        pltpu.async_copy(em_hbm.at[pl.ds(base + b * B, B)], emb[k], se[k])
        pltpu.async_copy(dst_hbm.at[pl.ds(base + b * B, B)], didxs.at[k],
                         si[k])

    def _compute(b, k):
        lo = b * B
        pltpu.make_async_copy(em_hbm.at[pl.ds(0, B)], emb[k], se[k]).wait()
        pltpu.make_async_copy(dst_hbm.at[pl.ds(0, B)], didxs.at[k],
                              si[k]).wait()

        def _edge(e, cc):
            cs = plsc.load_gather(coeffall, [jnp.full((L,), lo + e,
                                                      jnp.int32)])
            for ch in range(F // L):
                embufs[k, e, pl.ds(ch * L, L)] = (
                    embufs[k, e, pl.ds(ch * L, L)] * cs)
            return cc

        lax.fori_loop(0, B, _edge, 0)
        pltpu.async_copy(emb[k], h_sh.at[didxs.at[k]], ss[k], add=True)

    def _wait_scat(k):
        pltpu.make_async_copy(em_hbm.at[pl.ds(0, B)], emb[k], ss[k]).wait()

    _issue(0, 0)

    def _pair(j, carry):
        for k in range(2):
            b = 2 * j + k

            @pl.when(b >= 1)
            def _():
                _wait_scat(1 - k)

            @pl.when(b + 1 < NB)
            def _():
                _issue(b + 1, 1 - k)
            _compute(b, k)
        return carry

    lax.fori_loop(0, NB // 2, _pair, 0)
    _wait_scat(1)
    _compute(NB - 1, 0)
    _wait_scat(0)
    plsc.subcore_barrier()

    def _wc(i, cc):
        pltpu.sync_copy(h_sh.at[pl.ds(rows0 + i * B, B)],
                        hp_hbm.at[pl.ds(c * N + rows0 + i * B, B)])
        return cc
    lax.fori_loop(0, nch, _wc, 0)


def _sc_pass3(em, ee, src, dst, sm):
    kern = pl.kernel(
        _k3_body,
        out_type=jax.ShapeDtypeStruct((NC * N, F), jnp.float32),
        mesh=_mesh,
        compiler_params=pltpu.CompilerParams(needs_layout_passes=False),
        scratch_types=[pltpu.VMEM((CB,), jnp.int32),
                       pltpu.VMEM((CB,), jnp.float32),
                       pltpu.VMEM((EPW,), jnp.float32),
                       pltpu.VMEM((N,), jnp.float32),
                       pltpu.VMEM((2, B), jnp.int32),
                       pltpu.VMEM((2, B, F), jnp.float32),
                       pltpu.VMEM_SHARED((N, F), jnp.float32),
                       pltpu.SemaphoreType.DMA((2,)),
                       pltpu.SemaphoreType.DMA((2,)),
                       pltpu.SemaphoreType.DMA((2,))],
    )
    return kern(em, ee, src, dst, sm)


def kernel(input, edge, edge_embed, edge_list_nhop, edge_embed_nhop,
           confidence, entity_rank, Corpus_, a, a_2):
    x = input
    src = edge[0]
    dst = edge[1]
    w1 = a[:, :F].T
    w2 = a[:, F:2 * F].T
    w3 = a[:, 2 * F:].T
    a2v = a_2[0]
    p, q = _compute_pq(x, w1, w2)
    r = _compute_r(edge_embed, w3)
    em, ee, ersp = _sc_pass1(p, q, r, src, dst, a2v)
    ern, sm, ers = _sc_pass2(ersp, ee, src, dst, entity_rank)
    hp = _sc_pass3(em, ee, src, dst, sm)
    h = _finalize(hp[:N], hp[N:], ers[:, None])
    return (h, ern)


# speedup vs baseline: 1.2757x; 1.1740x over previous
import functools

import jax
import jax.numpy as jnp
from jax import lax
from jax.experimental import pallas as pl
from jax.experimental.pallas import tpu as pltpu
from jax.experimental.pallas import tpu_sc as plsc

ALPHA = 0.2
DAMP = 0.85
N = 10000
E = 320000
F = 128
NC = 2
NS = 16
NW = NC * NS
L = 16
EPW = E // NW
B = 80
NB = EPW // B
GPB = B // L
EPC = E // NS
CB = 2000
NCH = EPC // CB
RPT = N // NS
NP = 10240
BK = NP // NS
EPS = 1e-12

_mesh = plsc.VectorSubcoreMesh(
    core_axis_name="c", subcore_axis_name="s", num_cores=NC, num_subcores=NS)


def _pq_body(x_ref, w1_ref, w2_ref, p_ref, q_ref):
    xb = x_ref[...]
    p_ref[...] = jnp.dot(xb, w1_ref[...], preferred_element_type=jnp.float32)
    q_ref[...] = jnp.dot(xb, w2_ref[...], preferred_element_type=jnp.float32)


def _compute_pq(x, w1, w2):
    blk = 1000
    return pl.pallas_call(
        _pq_body,
        grid=(N // blk,),
        in_specs=[pl.BlockSpec((blk, F), lambda i: (i, 0)),
                  pl.BlockSpec((F, F), lambda i: (0, 0)),
                  pl.BlockSpec((F, F), lambda i: (0, 0))],
        out_specs=[pl.BlockSpec((blk, F), lambda i: (i, 0)),
                   pl.BlockSpec((blk, F), lambda i: (i, 0))],
        out_shape=[jax.ShapeDtypeStruct((N, F), jnp.float32)] * 2,
    )(x, w1, w2)


def _r_body(e_ref, w_ref, o_ref):
    o_ref[...] = jnp.dot(e_ref[...], w_ref[...],
                         preferred_element_type=jnp.float32)


def _compute_r(edge_embed, w3):
    blk = 4000
    nr = edge_embed.shape[1]
    return pl.pallas_call(
        _r_body,
        grid=(E // blk,),
        in_specs=[pl.BlockSpec((blk, nr), lambda i: (i, 0)),
                  pl.BlockSpec((nr, F), lambda i: (0, 0))],
        out_specs=pl.BlockSpec((blk, F), lambda i: (i, 0)),
        out_shape=jax.ShapeDtypeStruct((E, F), jnp.float32),
    )(edge_embed, w3)


def _fin_body(h0_ref, h1_ref, ers_ref, o_ref):
    hp = (h0_ref[...] + h1_ref[...]) / ers_ref[...]
    o_ref[...] = jnp.where(hp > 0.0, hp, jnp.exp(jnp.minimum(hp, 0.0)) - 1.0)


def _finalize(h0, h1, ers2d):
    blk = 1000
    return pl.pallas_call(
        _fin_body,
        grid=(N // blk,),
        in_specs=[pl.BlockSpec((blk, F), lambda i: (i, 0)),
                  pl.BlockSpec((blk, F), lambda i: (i, 0)),
                  pl.BlockSpec((blk, 1), lambda i: (i, 0))],
        out_specs=pl.BlockSpec((blk, F), lambda i: (i, 0)),
        out_shape=jax.ShapeDtypeStruct((N, F), jnp.float32),
    )(h0, h1, ers2d)


def _lane_total(v):
    idx = lax.broadcasted_iota(jnp.int32, (L,), 0)
    dnums = lax.GatherDimensionNumbers(
        offset_dims=(), collapsed_slice_dims=(0,), start_index_map=(0,))
    for k in (8, 4, 2, 1):
        perm = jnp.bitwise_xor(idx, k)
        v = v + lax.gather(v, perm[:, None], dnums, (1,),
                           mode=lax.GatherScatterMode.PROMISE_IN_BOUNDS)
    return v


def _zero_ref(ref, n):
    def _z(i, carry):
        ref[pl.ds(i * L, L)] = jnp.zeros((L,), jnp.float32)
        return carry
    lax.fori_loop(0, n // L, _z, 0)


def _acc_from(ref_dst, ref_src, n):
    def _a(i, carry):
        ref_dst[pl.ds(i * L, L)] = (ref_dst[pl.ds(i * L, L)]
                                    + ref_src[pl.ds(i * L, L)])
        return carry
    lax.fori_loop(0, n // L, _a, 0)


def _fix_zeros(ref, n):
    def _f(i, carry):
        v = ref[pl.ds(i * L, L)]
        ref[pl.ds(i * L, L)] = jnp.where(v == 0.0, EPS, v)
        return carry
    lax.fori_loop(0, n // L, _f, 0)


def _k1_body(p_hbm, q_hbm, r_hbm, src_hbm, dst_hbm, a2_hbm,
             em_hbm, ee_hbm, ers_hbm,
             srcall, dstall, eeall, erslocal, a2buf,
             pbufs, qbufs, rbufs, embufs,
             semp, semq, semr, seme):
    wid = lax.axis_index("s") * NC + lax.axis_index("c")
    base = wid * EPW
    pltpu.sync_copy(src_hbm.at[pl.ds(base, EPW)], srcall)
    pltpu.sync_copy(dst_hbm.at[pl.ds(base, EPW)], dstall)
    pltpu.sync_copy(a2_hbm, a2buf)
    _zero_ref(erslocal, NP)
    a2c = [a2buf[pl.ds(ch * L, L)] for ch in range(F // L)]
    pb = [pbufs.at[0], pbufs.at[1]]
    qb = [qbufs.at[0], qbufs.at[1]]
    rb = [rbufs.at[0], rbufs.at[1]]
    emb = [embufs.at[0], embufs.at[1]]
    sp = [semp.at[0], semp.at[1]]
    sq = [semq.at[0], semq.at[1]]
    sr = [semr.at[0], semr.at[1]]
    se = [seme.at[0], seme.at[1]]

    def _issue(b, k):
        lo = b * B
        pltpu.async_copy(p_hbm.at[srcall.at[pl.ds(lo, B)]], pb[k], sp[k])
        pltpu.async_copy(q_hbm.at[dstall.at[pl.ds(lo, B)]], qb[k], sq[k])
        pltpu.async_copy(r_hbm.at[pl.ds(base + lo, B)], rb[k], sr[k])

    def _wait_in(k):
        pltpu.make_async_copy(p_hbm.at[pl.ds(0, B)], pb[k], sp[k]).wait()
        pltpu.make_async_copy(q_hbm.at[pl.ds(0, B)], qb[k], sq[k]).wait()
        pltpu.make_async_copy(r_hbm.at[pl.ds(0, B)], rb[k], sr[k]).wait()

    def _compute(b, k):
        lo = b * B

        @functools.partial(plsc.parallel_loop, 0, B, unroll=4)
        def _edge(e):
            acc = jnp.zeros((L,), jnp.float32)
            for ch in range(F // L):
                pv = pb[k][e, pl.ds(ch * L, L)]
                qv = qb[k][e, pl.ds(ch * L, L)]
                rv = rb[k][e, pl.ds(ch * L, L)]
                m = pv + qv + rv
                emv = jnp.maximum(m, ALPHA * m)
                emb[k][e, pl.ds(ch * L, L)] = emv
                acc = acc + a2c[ch] * emv
            t = _lane_total(acc)
            plsc.store_scatter(eeall, [jnp.full((L,), lo + e, jnp.int32)], t)
        for g in range(GPB):
            t = eeall[pl.ds(lo + g * L, L)]
            powers = -jnp.maximum(t, ALPHA * t)
            ev = jnp.exp(powers)
            eeall[pl.ds(lo + g * L, L)] = ev
            dl = dstall[pl.ds(lo + g * L, L)]
            plsc.addupdate_scatter(erslocal, [dl], ev)
        pltpu.async_copy(emb[k], em_hbm.at[pl.ds(base + lo, B)], se[k])

    _issue(0, 0)
    _wait_in(0)

    def _pair(j, carry):
        for k in range(2):
            b = 2 * j + k

            @pl.when(b + 1 < NB)
            def _():
                _issue(b + 1, 1 - k)

            @pl.when(b >= 2)
            def _():
                pltpu.make_async_copy(p_hbm.at[pl.ds(0, B)], emb[k],
                                      se[k]).wait()
            _compute(b, k)

            @pl.when(b + 1 < NB)
            def _():
                _wait_in(1 - k)
        return carry

    lax.fori_loop(0, NB // 2, _pair, 0)
    pltpu.make_async_copy(p_hbm.at[pl.ds(0, B)], emb[0], se[0]).wait()
    _compute(NB - 1, 0)
    pltpu.make_async_copy(p_hbm.at[pl.ds(0, B)], emb[0], se[0]).wait()
    pltpu.make_async_copy(p_hbm.at[pl.ds(0, B)], emb[1], se[1]).wait()
    pltpu.sync_copy(eeall, ee_hbm.at[pl.ds(base, EPW)])
    pltpu.sync_copy(erslocal, ers_hbm.at[pl.ds(wid * NP, NP)])


def _sc_pass1(p, q, r, src, dst, a2v):
    kern = pl.kernel(
        _k1_body,
        out_type=[jax.ShapeDtypeStruct((E, F), jnp.float32),
                  jax.ShapeDtypeStruct((E,), jnp.float32),
                  jax.ShapeDtypeStruct((NW * NP,), jnp.float32)],
        mesh=_mesh,
        compiler_params=pltpu.CompilerParams(needs_layout_passes=False),
        scratch_types=[pltpu.VMEM((EPW,), jnp.int32),
                       pltpu.VMEM((EPW,), jnp.int32),
                       pltpu.VMEM((EPW,), jnp.float32),
                       pltpu.VMEM((NP,), jnp.float32),
                       pltpu.VMEM((F,), jnp.float32),
                       pltpu.VMEM((2, B, F), jnp.float32),
                       pltpu.VMEM((2, B, F), jnp.float32),
                       pltpu.VMEM((2, B, F), jnp.float32),
                       pltpu.VMEM((2, B, F), jnp.float32),
                       pltpu.SemaphoreType.DMA((2,)),
                       pltpu.SemaphoreType.DMA((2,)),
                       pltpu.SemaphoreType.DMA((2,)),
                       pltpu.SemaphoreType.DMA((2,))],
    )
    return kern(p, q, r, src, dst, a2v)


def _acc_rows(dst, ref2d, nrows, n):
    def _a(i, carry):
        v = ref2d[0, pl.ds(i * L, L)]
        for j in range(1, nrows):
            v = v + ref2d[j, pl.ds(i * L, L)]
        dst[pl.ds(i * L, L)] = v
        return carry
    lax.fori_loop(0, n // L, _a, 0)


def _k2_body(ersp_hbm, ee_hbm, src_hbm, dst_hbm, er_hbm,
             ern_hbm, sm_hbm, ers_hbm,
             ers, erb, den, num, smb, blk, srcb, dstb, eb, cb640,
             sharedA, sharedB, semb):
    c = lax.axis_index("c")
    s = lax.axis_index("s")
    pltpu.sync_copy(er_hbm, erb)

    for j in range(NW):
        pltpu.async_copy(ersp_hbm.at[pl.ds(j * NP + s * BK, BK)], blk.at[j],
                         semb)
    for j in range(NW):
        pltpu.make_async_copy(ersp_hbm.at[pl.ds(0, BK)], blk.at[j],
                              semb).wait()
    _acc_rows(cb640, blk, NW, BK)
    _fix_zeros(cb640, BK)
    pltpu.sync_copy(cb640, sharedB.at[pl.ds(s * BK, BK)])
    plsc.subcore_barrier()
    pltpu.sync_copy(sharedB, ers)
    plsc.subcore_barrier()

    ebase = s * EPC
    _zero_ref(den, NP)

    def _chunk_a(k, carry):
        off = ebase + k * CB
        pltpu.sync_copy(src_hbm.at[pl.ds(off, CB)], srcb)
        pltpu.sync_copy(ee_hbm.at[pl.ds(off, CB)], eb)

        def _grp(g, cc):
            sl = srcb[pl.ds(g * L, L)]
            ev = eb[pl.ds(g * L, L)]
            esrc = plsc.load_gather(ers, [sl])
            rav = ev / esrc
            plsc.addupdate_scatter(den, [sl], rav)
            return cc

        lax.fori_loop(0, CB // L, _grp, 0)
        return carry

    lax.fori_loop(0, NCH, _chunk_a, 0)

    pltpu.sync_copy(den, sharedA.at[pl.ds(s * NP, NP)])
    plsc.subcore_barrier()
    for j in range(NS):
        pltpu.async_copy(sharedA.at[pl.ds(j * NP + s * BK, BK)], blk.at[j],
                         semb)
    for j in range(NS):
        pltpu.make_async_copy(ersp_hbm.at[pl.ds(0, BK)], blk.at[j],
                              semb).wait()
    _acc_rows(cb640, blk, NS, BK)
    _fix_zeros(cb640, BK)
    pltpu.sync_copy(cb640, sharedB.at[pl.ds(s * BK, BK)])
    plsc.subcore_barrier()
    pltpu.sync_copy(sharedB, den)
    plsc.subcore_barrier()

    _zero_ref(num, NP)

    def _chunk_b(k, carry):
        off = ebase + k * CB
        pltpu.sync_copy(src_hbm.at[pl.ds(off, CB)], srcb)
        pltpu.sync_copy(dst_hbm.at[pl.ds(off, CB)], dstb)
        pltpu.sync_copy(ee_hbm.at[pl.ds(off, CB)], eb)

        def _grp(g, cc):
            sl = srcb[pl.ds(g * L, L)]
            dl = dstb[pl.ds(g * L, L)]
            ev = eb[pl.ds(g * L, L)]
            esrc = plsc.load_gather(ers, [sl])
            rav = ev / esrc
            densrc = plsc.load_gather(den, [sl])
            ersrc = plsc.load_gather(erb, [sl])
            nv = rav * ersrc / densrc
            plsc.addupdate_scatter(num, [dl], nv)
            return cc

        lax.fori_loop(0, CB // L, _grp, 0)
        return carry

    lax.fori_loop(0, NCH, _chunk_b, 0)

    pltpu.sync_copy(num, sharedA.at[pl.ds(s * NP, NP)])
    plsc.subcore_barrier()
    for j in range(NS):
        pltpu.async_copy(sharedA.at[pl.ds(j * NP + s * BK, BK)], blk.at[j],
                         semb)
    for j in range(NS):
        pltpu.make_async_copy(ersp_hbm.at[pl.ds(0, BK)], blk.at[j],
                              semb).wait()
    _acc_rows(cb640, blk, NS, BK)
    pltpu.sync_copy(cb640, sharedB.at[pl.ds(s * BK, BK)])
    plsc.subcore_barrier()
    pltpu.sync_copy(sharedB, num)

    def _ern(i, carry):
        v = num[pl.ds(i * L, L)]
        num[pl.ds(i * L, L)] = (1.0 - DAMP) + DAMP * v
        return carry
    lax.fori_loop(0, N // L, _ern, 0)

    def _mx(i, acc):
        return jnp.maximum(acc, num[pl.ds(i * L, L)])
    mx = jnp.max(lax.fori_loop(0, N // L, _mx,
                               jnp.full((L,), -jnp.inf, jnp.float32)))

    def _se(i, acc):
        return acc + jnp.exp(num[pl.ds(i * L, L)] - mx)
    ssum = jnp.sum(lax.fori_loop(0, N // L, _se,
                                 jnp.zeros((L,), jnp.float32)))

    def _sm(i, carry):
        smb[pl.ds(i * L, L)] = jnp.exp(num[pl.ds(i * L, L)] - mx) / ssum
        return carry
    lax.fori_loop(0, N // L, _sm, 0)

    @pl.when(jnp.logical_and(c == 0, s == 0))
    def _write():
        pltpu.sync_copy(num.at[pl.ds(0, N)], ern_hbm)
        pltpu.sync_copy(smb, sm_hbm)
        pltpu.sync_copy(ers.at[pl.ds(0, N)], ers_hbm)


def _sc_pass2(ersp, ee, src, dst, entity_rank):
    kern = pl.kernel(
        _k2_body,
        out_type=[jax.ShapeDtypeStruct((N,), jnp.float32),
                  jax.ShapeDtypeStruct((N,), jnp.float32),
                  jax.ShapeDtypeStruct((N,), jnp.float32)],
        mesh=_mesh,
        compiler_params=pltpu.CompilerParams(needs_layout_passes=False),
        scratch_types=[pltpu.VMEM((NP,), jnp.float32),
                       pltpu.VMEM((N,), jnp.float32),
                       pltpu.VMEM((NP,), jnp.float32),
                       pltpu.VMEM((NP,), jnp.float32),
                       pltpu.VMEM((N,), jnp.float32),
                       pltpu.VMEM((NW, BK), jnp.float32),
                       pltpu.VMEM((CB,), jnp.int32),
                       pltpu.VMEM((CB,), jnp.int32),
                       pltpu.VMEM((CB,), jnp.float32),
                       pltpu.VMEM((BK,), jnp.float32),
                       pltpu.VMEM_SHARED((NS * NP,), jnp.float32),
                       pltpu.VMEM_SHARED((NP,), jnp.float32),
                       pltpu.SemaphoreType.DMA],
    )
    return kern(ersp, ee, src, dst, entity_rank)


def _k3_body(em_hbm, ee_hbm, src_hbm, dst_hbm, sm_hbm,
             hp_hbm,
             srcb, eb, coeffall, smb, didxs, embufs, h_sh,
             seme, semi, sems):
    c = lax.axis_index("c")
    s = lax.axis_index("s")
    wid = s * NC + c
    base = wid * EPW
    pltpu.sync_copy(sm_hbm, smb)

    def _cchunk(k2, carry):
        off = k2 * CB
        pltpu.sync_copy(src_hbm.at[pl.ds(base + off, CB)], srcb)
        pltpu.sync_copy(ee_hbm.at[pl.ds(base + off, CB)], eb)

        def _grp(g, cc):
            sl = srcb[pl.ds(g * L, L)]
            smv = plsc.load_gather(smb, [sl])
            ev = eb[pl.ds(g * L, L)]
            coeffall[pl.ds(off + g * L, L)] = smv * ev
            return cc

        lax.fori_loop(0, CB // L, _grp, 0)
        return carry

    lax.fori_loop(0, EPW // CB, _cchunk, 0)

    def _zz(e, cc):
        for ch in range(F // L):
            embufs[0, e, pl.ds(ch * L, L)] = jnp.zeros((L,), jnp.float32)
        return cc
    lax.fori_loop(0, B, _zz, 0)
    rows0 = s * 640
    nch = jnp.where(s == NS - 1, 5, 8)

    def _zc(i, cc):
        pltpu.sync_copy(embufs.at[0], h_sh.at[pl.ds(rows0 + i * B, B)])
        return cc
    lax.fori_loop(0, nch, _zc, 0)
    plsc.subcore_barrier()

    emb = [embufs.at[0], embufs.at[1]]
    se = [seme.at[0], seme.at[1]]
    si = [semi.at[0], semi.at[1]]
    ss = [sems.at[0], sems.at[1]]

    def _issue(b, k):
        pltpu.async_copy(em_hbm.at[pl.ds(base + b * B, B)], emb[k], se[k])
        pltpu.async_copy(dst_hbm.at[pl.ds(base + b * B, B)], didxs.at[k],
                         si[k])

    def _compute(b, k):
        lo = b * B
        pltpu.make_async_copy(em_hbm.at[pl.ds(0, B)], emb[k], se[k]).wait()
        pltpu.make_async_copy(dst_hbm.at[pl.ds(0, B)], didxs.at[k],
                              si[k]).wait()

        @functools.partial(plsc.parallel_loop, 0, B, unroll=4)
        def _edge(e):
            cs = plsc.load_gather(coeffall, [jnp.full((L,), lo + e,
                                                      jnp.int32)])
            for ch in range(F // L):
                embufs[k, e, pl.ds(ch * L, L)] = (
                    embufs[k, e, pl.ds(ch * L, L)] * cs)
        pltpu.async_copy(emb[k], h_sh.at[didxs.at[k]], ss[k], add=True)

    def _wait_scat(k):
        pltpu.make_async_copy(em_hbm.at[pl.ds(0, B)], emb[k], ss[k]).wait()

    _issue(0, 0)

    def _pair(j, carry):
        for k in range(2):
            b = 2 * j + k

            @pl.when(b >= 1)
            def _():
                _wait_scat(1 - k)

            @pl.when(b + 1 < NB)
            def _():
                _issue(b + 1, 1 - k)
            _compute(b, k)
        return carry

    lax.fori_loop(0, NB // 2, _pair, 0)
    _wait_scat(1)
    _compute(NB - 1, 0)
    _wait_scat(0)
    plsc.subcore_barrier()

    def _wc(i, cc):
        pltpu.sync_copy(h_sh.at[pl.ds(rows0 + i * B, B)],
                        hp_hbm.at[pl.ds(c * N + rows0 + i * B, B)])
        return cc
    lax.fori_loop(0, nch, _wc, 0)


def _sc_pass3(em, ee, src, dst, sm):
    kern = pl.kernel(
        _k3_body,
        out_type=jax.ShapeDtypeStruct((NC * N, F), jnp.float32),
        mesh=_mesh,
        compiler_params=pltpu.CompilerParams(needs_layout_passes=False),
        scratch_types=[pltpu.VMEM((CB,), jnp.int32),
                       pltpu.VMEM((CB,), jnp.float32),
                       pltpu.VMEM((EPW,), jnp.float32),
                       pltpu.VMEM((N,), jnp.float32),
                       pltpu.VMEM((2, B), jnp.int32),
                       pltpu.VMEM((2, B, F), jnp.float32),
                       pltpu.VMEM_SHARED((N, F), jnp.float32),
                       pltpu.SemaphoreType.DMA((2,)),
                       pltpu.SemaphoreType.DMA((2,)),
                       pltpu.SemaphoreType.DMA((2,))],
    )
    return kern(em, ee, src, dst, sm)


def kernel(input, edge, edge_embed, edge_list_nhop, edge_embed_nhop,
           confidence, entity_rank, Corpus_, a, a_2):
    x = input
    src = edge[0]
    dst = edge[1]
    w1 = a[:, :F].T
    w2 = a[:, F:2 * F].T
    w3 = a[:, 2 * F:].T
    a2v = a_2[0]
    p, q = _compute_pq(x, w1, w2)
    r = _compute_r(edge_embed, w3)
    em, ee, ersp = _sc_pass1(p, q, r, src, dst, a2v)
    ern, sm, ers = _sc_pass2(ersp, ee, src, dst, entity_rank)
    hp = _sc_pass3(em, ee, src, dst, sm)
    h = _finalize(hp[:N], hp[N:], ers[:, None])
    return (h, ern)
